# baseline (speedup 1.0000x reference)
import functools

import jax
import jax.numpy as jnp
from jax import lax
from jax.experimental import pallas as pl
from jax.experimental.pallas import tpu as pltpu
from jax.experimental.pallas import tpu_sc as plsc

D = 128
DW = 144
NC, NS = 2, 16
NW = NC * NS
CHUNK = 128


def _sc_aggregate(ei, xw, n_pad, c_per_tile):
    rows_per_tile = n_pad // NS
    mesh = plsc.VectorSubcoreMesh(core_axis_name="c", subcore_axis_name="s")

    @functools.partial(
        pl.kernel,
        out_type=jax.ShapeDtypeStruct((NC, n_pad, DW), jnp.float32),
        mesh=mesh,
        compiler_params=pltpu.CompilerParams(use_tc_tiling_on_sc=False),
        scratch_types=[
            pltpu.VMEM((2, CHUNK), jnp.int32),
            pltpu.VMEM((CHUNK, DW), jnp.float32),
            pltpu.VMEM_SHARED((n_pad, DW), jnp.float32),
            pltpu.SemaphoreType.DMA,
        ],
    )
    def k(ei_hbm, xw_hbm, out_hbm, idx, rows, acc, sem):
        cid = lax.axis_index("c")
        sid = lax.axis_index("s")
        wid = cid * NS + sid

        @pl.loop(0, CHUNK)
        def _(r):
            @pl.loop(0, DW // 16)
            def _(cc):
                rows.at[pl.ds(r, 1), pl.ds(cc * 16, 16)][...] = (
                    jnp.zeros((1, 16), jnp.float32))

        @pl.loop(0, rows_per_tile // CHUNK)
        def _(kk):
            pltpu.sync_copy(
                rows, acc.at[pl.ds(sid * rows_per_tile + kk * CHUNK, CHUNK)])

        plsc.subcore_barrier()

        base = wid * c_per_tile

        @pl.loop(0, c_per_tile)
        def _(ci):
            pltpu.sync_copy(ei_hbm.at[base + ci], idx)
            pltpu.async_copy(xw_hbm.at[idx.at[0]], rows, sem).wait()
            pltpu.sync_copy(rows, acc.at[idx.at[1]], add=True)

        plsc.subcore_barrier()

        @pl.loop(0, rows_per_tile // CHUNK)
        def _(h):
            r0 = sid * rows_per_tile + h * CHUNK
            pltpu.sync_copy(acc.at[pl.ds(r0, CHUNK)], rows)
            pltpu.sync_copy(rows, out_hbm.at[cid, pl.ds(r0, CHUNK)])

    return k(ei, xw)


def _tc_update(partials, xw, wt, b2, n_pad):
    blk = 1024
    grid = n_pad // blk

    def body(p_ref, xw_ref, wt_ref, b_ref, o_ref):
        s = p_ref[0] + p_ref[1] + xw_ref[...]
        cnt = jnp.maximum(s[:, D:D + 1], 1.0)
        aggr = s[:, :D] / cnt
        out = jnp.dot(aggr, wt_ref[...],
                      preferred_element_type=jnp.float32) + b_ref[...]
        nrm = jnp.sqrt(jnp.sum(out * out, axis=1, keepdims=True))
        o_ref[...] = out / jnp.maximum(nrm, 1e-12)

    return pl.pallas_call(
        body,
        grid=(grid,),
        in_specs=[
            pl.BlockSpec((NC, blk, DW), lambda i: (0, i, 0)),
            pl.BlockSpec((blk, DW), lambda i: (i, 0)),
            pl.BlockSpec((D, D), lambda i: (0, 0)),
            pl.BlockSpec((1, D), lambda i: (0, 0)),
        ],
        out_specs=pl.BlockSpec((blk, D), lambda i: (i, 0)),
        out_shape=jax.ShapeDtypeStruct((n_pad, D), jnp.float32),
    )(partials, xw, wt, b2)


def kernel(x, edge_index, W, b):
    n = x.shape[0]
    e = edge_index.shape[1]
    n_pad = ((n + 1 + 2047) // 2048) * 2048
    c_per_tile = 2 * ((e + 2 * CHUNK * NW - 1) // (2 * CHUNK * NW))
    e_pad = c_per_tile * CHUNK * NW
    e_tile = c_per_tile * CHUNK
    pad = e_pad - e

    src = edge_index[0].astype(jnp.int32)
    dst = edge_index[1].astype(jnp.int32)
    if pad > 0:
        dummy = n + (jnp.arange(pad, dtype=jnp.int32) % (n_pad - 1 - n))
        if pad % NW == 0:
            pad_per_tile = pad // NW
            real_per_tile = e_tile - pad_per_tile
            src = jnp.concatenate(
                [src.reshape(NW, real_per_tile),
                 jnp.zeros((NW, pad_per_tile), jnp.int32)], axis=1)
            dst = jnp.concatenate(
                [dst.reshape(NW, real_per_tile),
                 dummy.reshape(NW, pad_per_tile)], axis=1)
        else:
            src = jnp.concatenate([src, jnp.zeros((pad,), jnp.int32)])
            dst = jnp.concatenate([dst, dummy])
    ei = jnp.stack([src.reshape(NW, c_per_tile, CHUNK),
                    dst.reshape(NW, c_per_tile, CHUNK)], axis=2)
    ei = ei.reshape(NW * c_per_tile, 2, CHUNK)

    xw = jnp.concatenate([x, jnp.ones((n, DW - D), jnp.float32)], axis=1)
    xw = jnp.pad(xw, ((0, n_pad - n), (0, 0)))

    partials = _sc_aggregate(ei, xw, n_pad, c_per_tile)
    out = _tc_update(partials, xw, W.T, b.reshape(1, D), n_pad)
    return out[:n]


# speedup vs baseline: 1.2754x; 1.2754x over previous
import functools

import jax
import jax.numpy as jnp
from jax import lax
from jax.experimental import pallas as pl
from jax.experimental.pallas import tpu as pltpu
from jax.experimental.pallas import tpu_sc as plsc

D = 128
DW = 144
NC, NS = 2, 16
NW = NC * NS
CHUNK = 128


def _sc_aggregate(xw, src, dst, n_pad, c_per_tile):
    rows_per_tile = n_pad // NS
    mesh = plsc.VectorSubcoreMesh(core_axis_name="c", subcore_axis_name="s")

    @functools.partial(
        pl.kernel,
        out_type=jax.ShapeDtypeStruct((NC, n_pad, DW), jnp.float32),
        mesh=mesh,
        compiler_params=pltpu.CompilerParams(use_tc_tiling_on_sc=False),
        scratch_types=[
            pltpu.VMEM((CHUNK,), jnp.int32),
            pltpu.VMEM((CHUNK,), jnp.int32),
            pltpu.VMEM((CHUNK, DW), jnp.float32),
            pltpu.VMEM_SHARED((n_pad, DW), jnp.float32),
            pltpu.SemaphoreType.DMA,
        ],
    )
    def k(xw_hbm, src_hbm, dst_hbm, out_hbm, idx_s, idx_d, rows, acc, sem):
        cid = lax.axis_index("c")
        sid = lax.axis_index("s")
        wid = cid * NS + sid

        @pl.loop(0, CHUNK)
        def _(r):
            @pl.loop(0, DW // 16)
            def _(cc):
                rows.at[pl.ds(r, 1), pl.ds(cc * 16, 16)][...] = (
                    jnp.zeros((1, 16), jnp.float32))

        @pl.loop(0, rows_per_tile // CHUNK)
        def _(kk):
            pltpu.sync_copy(
                rows, acc.at[pl.ds(sid * rows_per_tile + kk * CHUNK, CHUNK)])

        plsc.subcore_barrier()

        base = wid * (c_per_tile * CHUNK)

        @pl.loop(0, c_per_tile)
        def _(ci):
            off = base + ci * CHUNK
            pltpu.sync_copy(src_hbm.at[pl.ds(off, CHUNK)], idx_s)
            pltpu.sync_copy(dst_hbm.at[pl.ds(off, CHUNK)], idx_d)
            pltpu.async_copy(xw_hbm.at[idx_s], rows, sem).wait()
            pltpu.sync_copy(rows, acc.at[idx_d], add=True)

        plsc.subcore_barrier()

        @pl.loop(0, rows_per_tile // CHUNK)
        def _(h):
            r0 = sid * rows_per_tile + h * CHUNK
            pltpu.sync_copy(acc.at[pl.ds(r0, CHUNK)], rows)
            pltpu.sync_copy(rows, out_hbm.at[cid, pl.ds(r0, CHUNK)])

    return k(xw, src, dst)


def _tc_update(partials, xw, wt, b2, n_pad):
    blk = 1024
    grid = n_pad // blk

    def body(p_ref, xw_ref, wt_ref, b_ref, o_ref):
        s = p_ref[0] + p_ref[1] + xw_ref[...]
        cnt = jnp.maximum(s[:, D:D + 1], 1.0)
        aggr = s[:, :D] / cnt
        out = jnp.dot(aggr, wt_ref[...],
                      preferred_element_type=jnp.float32) + b_ref[...]
        nrm = jnp.sqrt(jnp.sum(out * out, axis=1, keepdims=True))
        o_ref[...] = out / jnp.maximum(nrm, 1e-12)

    return pl.pallas_call(
        body,
        grid=(grid,),
        in_specs=[
            pl.BlockSpec((NC, blk, DW), lambda i: (0, i, 0)),
            pl.BlockSpec((blk, DW), lambda i: (i, 0)),
            pl.BlockSpec((D, D), lambda i: (0, 0)),
            pl.BlockSpec((1, D), lambda i: (0, 0)),
        ],
        out_specs=pl.BlockSpec((blk, D), lambda i: (i, 0)),
        out_shape=jax.ShapeDtypeStruct((n_pad, D), jnp.float32),
    )(partials, xw, wt, b2)


def kernel(x, edge_index, W, b):
    n = x.shape[0]
    e = edge_index.shape[1]
    n_pad = ((n + 1 + 2047) // 2048) * 2048
    c_per_tile = (e + CHUNK * NW - 1) // (CHUNK * NW)
    e_pad = c_per_tile * CHUNK * NW
    e_tile = c_per_tile * CHUNK
    pad = e_pad - e

    src = edge_index[0].astype(jnp.int32)
    dst = edge_index[1].astype(jnp.int32)
    if pad > 0:
        dummy = n + (jnp.arange(pad, dtype=jnp.int32) % (n_pad - 1 - n))
        if pad % NW == 0:
            pad_per_tile = pad // NW
            real_per_tile = e_tile - pad_per_tile
            src = jnp.concatenate(
                [src.reshape(NW, real_per_tile),
                 jnp.zeros((NW, pad_per_tile), jnp.int32)], axis=1).reshape(-1)
            dst = jnp.concatenate(
                [dst.reshape(NW, real_per_tile),
                 dummy.reshape(NW, pad_per_tile)], axis=1).reshape(-1)
        else:
            src = jnp.concatenate([src, jnp.zeros((pad,), jnp.int32)])
            dst = jnp.concatenate([dst, dummy])

    xw = jnp.concatenate([x, jnp.ones((n, DW - D), jnp.float32)], axis=1)
    xw = jnp.pad(xw, ((0, n_pad - n), (0, 0)))

    partials = _sc_aggregate(xw, src, dst, n_pad, c_per_tile)
    out = _tc_update(partials, xw, W.T, b.reshape(1, D), n_pad)
    return out[:n]


# speedup vs baseline: 1.4772x; 1.1582x over previous
import functools

import jax
import jax.numpy as jnp
from jax import lax
from jax.experimental import pallas as pl
from jax.experimental.pallas import tpu as pltpu
from jax.experimental.pallas import tpu_sc as plsc

D = 128
DW = 144
NC, NS = 2, 16
NW = NC * NS
CHUNK = 128


def _sc_aggregate(xw, src, dst, n_pad, c_per_tile):
    rows_per_tile = n_pad // NS
    e_tile = c_per_tile * CHUNK
    mesh = plsc.VectorSubcoreMesh(core_axis_name="c", subcore_axis_name="s")

    @functools.partial(
        pl.kernel,
        out_type=jax.ShapeDtypeStruct((NC, n_pad, DW), jnp.float32),
        mesh=mesh,
        compiler_params=pltpu.CompilerParams(use_tc_tiling_on_sc=False),
        scratch_types=[
            pltpu.VMEM((e_tile,), jnp.int32),
            pltpu.VMEM((e_tile,), jnp.int32),
            pltpu.VMEM((CHUNK, DW), jnp.float32),
            pltpu.VMEM_SHARED((n_pad, DW), jnp.float32),
            pltpu.SemaphoreType.DMA,
        ],
    )
    def k(xw_hbm, src_hbm, dst_hbm, out_hbm, src_all, dst_all, rows, acc, sem):
        cid = lax.axis_index("c")
        sid = lax.axis_index("s")
        wid = cid * NS + sid

        @pl.loop(0, CHUNK)
        def _(r):
            @pl.loop(0, DW // 16)
            def _(cc):
                rows.at[pl.ds(r, 1), pl.ds(cc * 16, 16)][...] = (
                    jnp.zeros((1, 16), jnp.float32))

        @pl.loop(0, rows_per_tile // CHUNK)
        def _(kk):
            pltpu.sync_copy(
                rows, acc.at[pl.ds(sid * rows_per_tile + kk * CHUNK, CHUNK)])

        base = wid * e_tile
        pltpu.sync_copy(src_hbm.at[pl.ds(base, e_tile)], src_all)
        pltpu.sync_copy(dst_hbm.at[pl.ds(base, e_tile)], dst_all)

        plsc.subcore_barrier()

        @pl.loop(0, c_per_tile)
        def _(ci):
            pltpu.async_copy(
                xw_hbm.at[src_all.at[pl.ds(ci * CHUNK, CHUNK)]], rows, sem
            ).wait()
            pltpu.sync_copy(
                rows, acc.at[dst_all.at[pl.ds(ci * CHUNK, CHUNK)]], add=True)

        plsc.subcore_barrier()

        @pl.loop(0, rows_per_tile // CHUNK)
        def _(h):
            r0 = sid * rows_per_tile + h * CHUNK
            pltpu.sync_copy(acc.at[pl.ds(r0, CHUNK)], rows)
            pltpu.sync_copy(rows, out_hbm.at[cid, pl.ds(r0, CHUNK)])

    return k(xw, src, dst)


def _tc_update(partials, xw, wt, b2, n_pad):
    blk = 1024
    grid = n_pad // blk

    def body(p_ref, xw_ref, wt_ref, b_ref, o_ref):
        s = p_ref[0] + p_ref[1] + xw_ref[...]
        cnt = jnp.maximum(s[:, D:D + 1], 1.0)
        aggr = s[:, :D] / cnt
        out = jnp.dot(aggr, wt_ref[...],
                      preferred_element_type=jnp.float32) + b_ref[...]
        nrm = jnp.sqrt(jnp.sum(out * out, axis=1, keepdims=True))
        o_ref[...] = out / jnp.maximum(nrm, 1e-12)

    return pl.pallas_call(
        body,
        grid=(grid,),
        in_specs=[
            pl.BlockSpec((NC, blk, DW), lambda i: (0, i, 0)),
            pl.BlockSpec((blk, DW), lambda i: (i, 0)),
            pl.BlockSpec((D, D), lambda i: (0, 0)),
            pl.BlockSpec((1, D), lambda i: (0, 0)),
        ],
        out_specs=pl.BlockSpec((blk, D), lambda i: (i, 0)),
        out_shape=jax.ShapeDtypeStruct((n_pad, D), jnp.float32),
    )(partials, xw, wt, b2)


def kernel(x, edge_index, W, b):
    n = x.shape[0]
    e = edge_index.shape[1]
    n_pad = ((n + 1 + 2047) // 2048) * 2048
    c_per_tile = (e + CHUNK * NW - 1) // (CHUNK * NW)
    e_pad = c_per_tile * CHUNK * NW
    e_tile = c_per_tile * CHUNK
    pad = e_pad - e

    src = edge_index[0].astype(jnp.int32)
    dst = edge_index[1].astype(jnp.int32)
    if pad > 0:
        dummy = n + (jnp.arange(pad, dtype=jnp.int32) % (n_pad - 1 - n))
        if pad % NW == 0:
            pad_per_tile = pad // NW
            real_per_tile = e_tile - pad_per_tile
            src = jnp.concatenate(
                [src.reshape(NW, real_per_tile),
                 jnp.zeros((NW, pad_per_tile), jnp.int32)], axis=1).reshape(-1)
            dst = jnp.concatenate(
                [dst.reshape(NW, real_per_tile),
                 dummy.reshape(NW, pad_per_tile)], axis=1).reshape(-1)
        else:
            src = jnp.concatenate([src, jnp.zeros((pad,), jnp.int32)])
            dst = jnp.concatenate([dst, dummy])

    xw = jnp.concatenate([x, jnp.ones((n, DW - D), jnp.float32)], axis=1)
    xw = jnp.pad(xw, ((0, n_pad - n), (0, 0)))

    partials = _sc_aggregate(xw, src, dst, n_pad, c_per_tile)
    out = _tc_update(partials, xw, W.T, b.reshape(1, D), n_pad)
    return out[:n]


# speedup vs baseline: 1.6354x; 1.1071x over previous
import functools

import jax
import jax.numpy as jnp
from jax import lax
from jax.experimental import pallas as pl
from jax.experimental.pallas import tpu as pltpu
from jax.experimental.pallas import tpu_sc as plsc

D = 128
NC, NS = 2, 16
NW = NC * NS
CHUNK = 128


def _sc_aggregate(xp, src, dst, n_pad, c_per_tile):
    rows_per_tile = n_pad // NS
    e_tile = c_per_tile * CHUNK
    hist_rows = n_pad // D
    mesh = plsc.VectorSubcoreMesh(core_axis_name="c", subcore_axis_name="s")

    @functools.partial(
        pl.kernel,
        out_type=(
            jax.ShapeDtypeStruct((NC, n_pad, D), jnp.float32),
            jax.ShapeDtypeStruct((NC, NS, hist_rows, D), jnp.float32),
        ),
        mesh=mesh,
        compiler_params=pltpu.CompilerParams(
            use_tc_tiling_on_sc=False, needs_layout_passes=False),
        scratch_types=[
            pltpu.VMEM((e_tile,), jnp.int32),
            pltpu.VMEM((e_tile,), jnp.int32),
            pltpu.VMEM((CHUNK, D), jnp.float32),
            pltpu.VMEM((hist_rows, D), jnp.float32),
            pltpu.VMEM_SHARED((n_pad, D), jnp.float32),
            pltpu.SemaphoreType.DMA,
        ],
    )
    def k(xp_hbm, src_hbm, dst_hbm, out_hbm, cnt_hbm, src_all, dst_all, rows,
          hist, acc, sem):
        cid = lax.axis_index("c")
        sid = lax.axis_index("s")
        wid = cid * NS + sid

        pltpu.sync_copy(xp_hbm.at[pl.ds(n_pad - CHUNK, CHUNK)], rows)
        pltpu.sync_copy(xp_hbm.at[pl.ds(n_pad - hist_rows, hist_rows)], hist)

        @pl.loop(0, rows_per_tile // CHUNK)
        def _(kk):
            pltpu.sync_copy(
                rows, acc.at[pl.ds(sid * rows_per_tile + kk * CHUNK, CHUNK)])

        base = wid * e_tile
        pltpu.sync_copy(src_hbm.at[pl.ds(base, e_tile)], src_all)
        pltpu.sync_copy(dst_hbm.at[pl.ds(base, e_tile)], dst_all)

        plsc.subcore_barrier()

        @pl.loop(0, c_per_tile)
        def _(ci):
            pltpu.async_copy(
                xp_hbm.at[src_all.at[pl.ds(ci * CHUNK, CHUNK)]], rows, sem
            ).wait()
            pltpu.sync_copy(
                rows, acc.at[dst_all.at[pl.ds(ci * CHUNK, CHUNK)]], add=True)

        ones16 = jnp.ones((16,), jnp.float32)

        @pl.loop(0, e_tile // 16)
        def _(j):
            dv = dst_all[pl.ds(j * 16, 16)]
            ri = lax.shift_right_logical(dv, 7)
            ci2 = lax.bitwise_and(dv, 127)
            plsc.addupdate_scatter(hist, [ri, ci2], ones16)

        pltpu.sync_copy(hist, cnt_hbm.at[cid, sid])

        plsc.subcore_barrier()

        @pl.loop(0, rows_per_tile // CHUNK)
        def _(h):
            r0 = sid * rows_per_tile + h * CHUNK
            pltpu.sync_copy(acc.at[pl.ds(r0, CHUNK)], rows)
            pltpu.sync_copy(rows, out_hbm.at[cid, pl.ds(r0, CHUNK)])

    return k(xp, src, dst)


def _tc_update(partials, counts, xp, wt, b2, n_pad):
    blk = 1024
    grid = n_pad // blk

    def body(p_ref, c_ref, x_ref, wt_ref, b_ref, o_ref):
        s = p_ref[0] + p_ref[1] + x_ref[...]
        cnt = lax.dot_general(
            c_ref[...], jnp.ones((NW, 1), jnp.float32),
            ((( 0,), (0,)), ((), ())),
            preferred_element_type=jnp.float32) + 1.0
        aggr = s / jnp.maximum(cnt, 1.0)
        out = jnp.dot(aggr, wt_ref[...],
                      preferred_element_type=jnp.float32) + b_ref[...]
        nrm = jnp.sqrt(jnp.sum(out * out, axis=1, keepdims=True))
        o_ref[...] = out / jnp.maximum(nrm, 1e-12)

    return pl.pallas_call(
        body,
        grid=(grid,),
        in_specs=[
            pl.BlockSpec((NC, blk, D), lambda i: (0, i, 0)),
            pl.BlockSpec((NW, blk), lambda i: (0, i)),
            pl.BlockSpec((blk, D), lambda i: (i, 0)),
            pl.BlockSpec((D, D), lambda i: (0, 0)),
            pl.BlockSpec((1, D), lambda i: (0, 0)),
        ],
        out_specs=pl.BlockSpec((blk, D), lambda i: (i, 0)),
        out_shape=jax.ShapeDtypeStruct((n_pad, D), jnp.float32),
    )(partials, counts, xp, wt, b2)


def kernel(x, edge_index, W, b):
    n = x.shape[0]
    e = edge_index.shape[1]
    n_pad = ((n + 1 + 2047) // 2048) * 2048
    c_per_tile = (e + CHUNK * NW - 1) // (CHUNK * NW)
    e_pad = c_per_tile * CHUNK * NW
    e_tile = c_per_tile * CHUNK
    pad = e_pad - e

    src = edge_index[0].astype(jnp.int32)
    dst = edge_index[1].astype(jnp.int32)
    if pad > 0:
        dummy = n + (jnp.arange(pad, dtype=jnp.int32) % (n_pad - 1 - n))
        if pad % NW == 0:
            pad_per_tile = pad // NW
            real_per_tile = e_tile - pad_per_tile
            src = jnp.concatenate(
                [src.reshape(NW, real_per_tile),
                 jnp.zeros((NW, pad_per_tile), jnp.int32)], axis=1).reshape(-1)
            dst = jnp.concatenate(
                [dst.reshape(NW, real_per_tile),
                 dummy.reshape(NW, pad_per_tile)], axis=1).reshape(-1)
        else:
            src = jnp.concatenate([src, jnp.zeros((pad,), jnp.int32)])
            dst = jnp.concatenate([dst, dummy])

    xp = jnp.pad(x, ((0, n_pad - n), (0, 0)))

    partials, counts = _sc_aggregate(xp, src, dst, n_pad, c_per_tile)
    counts = counts.reshape(NW, n_pad)
    out = _tc_update(partials, counts, xp, W.T, b.reshape(1, D), n_pad)
    return out[:n]


# speedup vs baseline: 1.6502x; 1.0091x over previous
import functools

import jax
import jax.numpy as jnp
from jax import lax
from jax.experimental import pallas as pl
from jax.experimental.pallas import tpu as pltpu
from jax.experimental.pallas import tpu_sc as plsc

D = 128
NC, NS = 2, 16
NW = NC * NS
CHUNK = 128


def _sc_aggregate(xp, src, dst, n_pad, c_per_tile):
    rows_per_tile = n_pad // NS
    e_tile = c_per_tile * CHUNK
    hist_rows = n_pad // D
    mesh = plsc.VectorSubcoreMesh(core_axis_name="c", subcore_axis_name="s")

    @functools.partial(
        pl.kernel,
        out_type=(
            jax.ShapeDtypeStruct((NC, n_pad, D), jnp.float32),
            jax.ShapeDtypeStruct((NC, NS, hist_rows, D), jnp.float32),
        ),
        mesh=mesh,
        compiler_params=pltpu.CompilerParams(
            use_tc_tiling_on_sc=False, needs_layout_passes=False),
        scratch_types=[
            pltpu.VMEM((e_tile,), jnp.int32),
            pltpu.VMEM((e_tile,), jnp.int32),
            pltpu.VMEM((CHUNK, D), jnp.float32),
            pltpu.VMEM((hist_rows, D), jnp.float32),
            pltpu.VMEM_SHARED((n_pad, D), jnp.float32),
            pltpu.SemaphoreType.DMA,
        ],
    )
    def k(xp_hbm, src_hbm, dst_hbm, out_hbm, cnt_hbm, src_all, dst_all, rows,
          hist, acc, sem):
        cid = lax.axis_index("c")
        sid = lax.axis_index("s")
        wid = cid * NS + sid

        pltpu.sync_copy(xp_hbm.at[pl.ds(n_pad - CHUNK, CHUNK)], rows)
        pltpu.sync_copy(xp_hbm.at[pl.ds(n_pad - hist_rows, hist_rows)], hist)

        @pl.loop(0, rows_per_tile // CHUNK)
        def _(kk):
            pltpu.sync_copy(
                rows, acc.at[pl.ds(sid * rows_per_tile + kk * CHUNK, CHUNK)])

        base = wid * e_tile
        pltpu.sync_copy(src_hbm.at[pl.ds(base, e_tile)], src_all)
        pltpu.sync_copy(dst_hbm.at[pl.ds(base, e_tile)], dst_all)

        plsc.subcore_barrier()

        ones16 = jnp.ones((16,), jnp.float32)

        @pl.loop(0, c_per_tile)
        def _(ci):
            h = pltpu.async_copy(
                xp_hbm.at[src_all.at[pl.ds(ci * CHUNK, CHUNK)]], rows, sem)
            for j in range(CHUNK // 16):
                dv = dst_all[pl.ds(ci * CHUNK + j * 16, 16)]
                ri = lax.shift_right_logical(dv, 7)
                ci2 = lax.bitwise_and(dv, 127)
                plsc.addupdate_scatter(hist, [ri, ci2], ones16)
            h.wait()
            pltpu.sync_copy(
                rows, acc.at[dst_all.at[pl.ds(ci * CHUNK, CHUNK)]], add=True)

        pltpu.sync_copy(hist, cnt_hbm.at[cid, sid])

        plsc.subcore_barrier()

        @pl.loop(0, rows_per_tile // CHUNK)
        def _(h):
            r0 = sid * rows_per_tile + h * CHUNK
            pltpu.sync_copy(acc.at[pl.ds(r0, CHUNK)], rows)
            pltpu.sync_copy(rows, out_hbm.at[cid, pl.ds(r0, CHUNK)])

    return k(xp, src, dst)


def _tc_update(partials, counts, xp, wt, b2, n_pad):
    blk = 1024
    grid = n_pad // blk

    def body(p_ref, c_ref, x_ref, wt_ref, b_ref, o_ref):
        s = p_ref[0] + p_ref[1] + x_ref[...]
        cnt = lax.dot_general(
            c_ref[...], jnp.ones((NW, 1), jnp.float32),
            ((( 0,), (0,)), ((), ())),
            preferred_element_type=jnp.float32) + 1.0
        aggr = s / jnp.maximum(cnt, 1.0)
        out = jnp.dot(aggr, wt_ref[...],
                      preferred_element_type=jnp.float32) + b_ref[...]
        nrm = jnp.sqrt(jnp.sum(out * out, axis=1, keepdims=True))
        o_ref[...] = out / jnp.maximum(nrm, 1e-12)

    return pl.pallas_call(
        body,
        grid=(grid,),
        in_specs=[
            pl.BlockSpec((NC, blk, D), lambda i: (0, i, 0)),
            pl.BlockSpec((NW, blk), lambda i: (0, i)),
            pl.BlockSpec((blk, D), lambda i: (i, 0)),
            pl.BlockSpec((D, D), lambda i: (0, 0)),
            pl.BlockSpec((1, D), lambda i: (0, 0)),
        ],
        out_specs=pl.BlockSpec((blk, D), lambda i: (i, 0)),
        out_shape=jax.ShapeDtypeStruct((n_pad, D), jnp.float32),
    )(partials, counts, xp, wt, b2)


def kernel(x, edge_index, W, b):
    n = x.shape[0]
    e = edge_index.shape[1]
    n_pad = ((n + 1 + 2047) // 2048) * 2048
    c_per_tile = (e + CHUNK * NW - 1) // (CHUNK * NW)
    e_pad = c_per_tile * CHUNK * NW
    e_tile = c_per_tile * CHUNK
    pad = e_pad - e

    src = edge_index[0].astype(jnp.int32)
    dst = edge_index[1].astype(jnp.int32)
    if pad > 0:
        dummy = n + (jnp.arange(pad, dtype=jnp.int32) % (n_pad - 1 - n))
        if pad % NW == 0:
            pad_per_tile = pad // NW
            real_per_tile = e_tile - pad_per_tile
            src = jnp.concatenate(
                [src.reshape(NW, real_per_tile),
                 jnp.zeros((NW, pad_per_tile), jnp.int32)], axis=1).reshape(-1)
            dst = jnp.concatenate(
                [dst.reshape(NW, real_per_tile),
                 dummy.reshape(NW, pad_per_tile)], axis=1).reshape(-1)
        else:
            src = jnp.concatenate([src, jnp.zeros((pad,), jnp.int32)])
            dst = jnp.concatenate([dst, dummy])

    xp = jnp.pad(x, ((0, n_pad - n), (0, 0)))

    partials, counts = _sc_aggregate(xp, src, dst, n_pad, c_per_tile)
    counts = counts.reshape(NW, n_pad)
    out = _tc_update(partials, counts, xp, W.T, b.reshape(1, D), n_pad)
    return out[:n]
